# baseline (speedup 1.0000x reference)
import functools

import jax
import jax.numpy as jnp
from jax import lax
from jax.experimental import pallas as pl
from jax.experimental.pallas import tpu as pltpu
from jax.experimental.pallas import tpu_sc as plsc

N = 10000
E = 320000
D = 128
NC = 2
NS = 16
NW = NC * NS
B = 64
NBUF = 3
NB = 162
EPT = NB * B
EPAD = EPT * NW
ROW_CHUNK = 632


def _sc_spmm_body(src_hbm, dst_hbm, w_hbm, x_hbm, zeros_hbm, out_hbm,
                  sb0, sb1, sb2, db0, db1, db2, wb0, wb1, wb2,
                  dc0, dc1, dc2, rows0, rows1, rows2, agg_sh,
                  is0, is1, is2, gs0, gs1, gs2, ss0, ss1, ss2):
    sbuf = (sb0, sb1, sb2)
    dbuf = (db0, db1, db2)
    wbuf = (wb0, wb1, wb2)
    dstc = (dc0, dc1, dc2)
    rows = (rows0, rows1, rows2)
    isem = (is0, is1, is2)
    gsem = (gs0, gs1, gs2)
    ssem = (ss0, ss1, ss2)
    c = lax.axis_index("c")
    s = lax.axis_index("s")
    wid = s * NC + c

    row_off = s * ROW_CHUNK
    last_off = (NS - 1) * ROW_CHUNK
    last_cnt = N - last_off

    @pl.when(s < NS - 1)
    def _zero_main():
        pltpu.sync_copy(zeros_hbm.at[pl.ds(row_off, ROW_CHUNK)],
                        agg_sh.at[pl.ds(row_off, ROW_CHUNK)])

    @pl.when(s == NS - 1)
    def _zero_last():
        pltpu.sync_copy(zeros_hbm.at[pl.ds(last_off, last_cnt)],
                        agg_sh.at[pl.ds(last_off, last_cnt)])

    plsc.subcore_barrier()

    base = wid * EPT

    def _stage(b, j):
        off = base + b * B
        pltpu.async_copy(src_hbm.at[pl.ds(off, B)], sbuf[j], isem[j])
        pltpu.async_copy(dst_hbm.at[pl.ds(off, B)], dbuf[j], isem[j])
        pltpu.async_copy(w_hbm.at[pl.ds(off, B)], wbuf[j], isem[j])

    def _stage_wait(b, j):
        off = base + b * B
        pltpu.make_async_copy(src_hbm.at[pl.ds(off, B)], sbuf[j],
                              isem[j]).wait()
        pltpu.make_async_copy(dst_hbm.at[pl.ds(off, B)], dbuf[j],
                              isem[j]).wait()
        pltpu.make_async_copy(w_hbm.at[pl.ds(off, B)], wbuf[j],
                              isem[j]).wait()

    for j in range(NBUF):
        _stage(j, j)
    for j in range(2):
        _stage_wait(j, j)
        pltpu.async_copy(x_hbm.at[sbuf[j]], rows[j], gsem[j])

    def outer(tt, carry):
        for j in range(NBUF):
            b = tt * NBUF + j
            pltpu.make_async_copy(x_hbm.at[sbuf[j]], rows[j],
                                  gsem[j]).wait()

            def group_body(g, carry2, j=j):
                w16 = wbuf[j][pl.ds(g * 16, 16)]
                dstc[j][pl.ds(g * 16, 16)] = dbuf[j][pl.ds(g * 16, 16)]
                for e16 in range(16):
                    wsp = jnp.full((16,), w16[e16], jnp.float32)
                    for f in range(D // 16):
                        rows[j][g * 16 + e16, pl.ds(f * 16, 16)] = (
                            rows[j][g * 16 + e16, pl.ds(f * 16, 16)] * wsp)
                return carry2

            lax.fori_loop(0, B // 16, group_body, 0)

            pltpu.async_copy(rows[j], agg_sh.at[dstc[j]], ssem[j], add=True)

            @pl.when(b + NBUF < NB)
            def _stage_next(j=j, b=b):
                _stage(b + NBUF, j)

            b2 = b + 2
            j2 = (j + 2) % NBUF

            @pl.when((b >= 1) & (b2 < NB))
            def _drain_prev(j2=j2):
                pltpu.make_async_copy(rows[j2], agg_sh.at[dstc[j2]],
                                      ssem[j2]).wait()

            @pl.when(b2 < NB)
            def _gather_next(j2=j2, b2=b2):
                _stage_wait(b2, j2)
                pltpu.async_copy(x_hbm.at[sbuf[j2]], rows[j2], gsem[j2])
        return carry

    lax.fori_loop(0, NB // NBUF, outer, 0)

    for j in range(NBUF):
        pltpu.make_async_copy(rows[j], agg_sh.at[dstc[j]], ssem[j]).wait()

    plsc.subcore_barrier()

    @pl.when(s < NS - 1)
    def _out_main():
        pltpu.sync_copy(agg_sh.at[pl.ds(row_off, ROW_CHUNK)],
                        out_hbm.at[c, pl.ds(row_off, ROW_CHUNK)])

    @pl.when(s == NS - 1)
    def _out_last():
        pltpu.sync_copy(agg_sh.at[pl.ds(last_off, last_cnt)],
                        out_hbm.at[c, pl.ds(last_off, last_cnt)])


_sc_spmm = functools.partial(
    pl.kernel,
    out_type=jax.ShapeDtypeStruct((NC, N, D), jnp.float32),
    mesh=plsc.VectorSubcoreMesh(core_axis_name="c", subcore_axis_name="s"),
    scratch_types=(
        [pltpu.VMEM((B,), jnp.int32) for _ in range(2 * NBUF)]
        + [pltpu.VMEM((B,), jnp.float32) for _ in range(NBUF)]
        + [pltpu.VMEM((B,), jnp.int32) for _ in range(NBUF)]
        + [pltpu.VMEM((B, D), jnp.float32) for _ in range(NBUF)]
        + [pltpu.VMEM_SHARED((N, D), jnp.float32)]
        + [pltpu.SemaphoreType.DMA for _ in range(3 * NBUF)]
    ),
)(_sc_spmm_body)


def _tc_body(agg_ref, x_ref, w_ref, sw_ref, bias_ref, gamma_ref, beta_ref,
             out_ref):
    a = agg_ref[0] + agg_ref[1]
    pre = jnp.dot(a, w_ref[...], preferred_element_type=jnp.float32)
    pre = pre + jnp.dot(x_ref[...], sw_ref[...],
                        preferred_element_type=jnp.float32)
    pre = pre + bias_ref[...]
    mean = jnp.mean(pre, axis=0, keepdims=True)
    cen = pre - mean
    var = jnp.mean(cen * cen, axis=0, keepdims=True)
    out_ref[...] = cen * lax.rsqrt(var + 1e-5) * gamma_ref[...] + beta_ref[...]


def kernel(x, edge_weight, weight, self_weight, bias, gamma, beta, edge_index):
    pad = EPAD - E
    dst = jnp.concatenate([edge_index[0], jnp.zeros((pad,), jnp.int32)])
    src = jnp.concatenate([edge_index[1], jnp.zeros((pad,), jnp.int32)])
    w = jnp.concatenate([edge_weight, jnp.zeros((pad,), jnp.float32)])
    zeros = jnp.zeros((N, D), jnp.float32)

    agg = _sc_spmm(src, dst, w, x, zeros)

    out = pl.pallas_call(
        _tc_body,
        out_shape=jax.ShapeDtypeStruct((N, D), jnp.float32),
    )(agg, x, weight, self_weight,
      bias.reshape(1, D), gamma.reshape(1, D), beta.reshape(1, D))
    return out


# speedup vs baseline: 1.5251x; 1.5251x over previous
import functools

import jax
import jax.numpy as jnp
from jax import lax
from jax.experimental import pallas as pl
from jax.experimental.pallas import tpu as pltpu
from jax.experimental.pallas import tpu_sc as plsc

N = 10000
E = 320000
D = 128
NC = 2
NS = 16
NW = NC * NS
B = 96
NBUF = 2
NB = 106
EPT = NB * B
EPAD = EPT * NW
ROW_CHUNK = 632


def _sc_spmm_body(sd_hbm, w_hbm, x_hbm, zeros_hbm, out_hbm,
                  sd0, sd1, wb0, wb1, dc0, dc1, rows0, rows1, agg_sh,
                  is0, is1, gs0, gs1, ss0, ss1):
    sdbuf = (sd0, sd1)
    wbuf = (wb0, wb1)
    dstc = (dc0, dc1)
    rows = (rows0, rows1)
    isem = (is0, is1)
    gsem = (gs0, gs1)
    ssem = (ss0, ss1)
    c = lax.axis_index("c")
    s = lax.axis_index("s")
    wid = s * NC + c

    row_off = s * ROW_CHUNK
    last_off = (NS - 1) * ROW_CHUNK
    last_cnt = N - last_off

    @pl.when(s < NS - 1)
    def _zero_main():
        pltpu.sync_copy(zeros_hbm.at[pl.ds(row_off, ROW_CHUNK)],
                        agg_sh.at[pl.ds(row_off, ROW_CHUNK)])

    @pl.when(s == NS - 1)
    def _zero_last():
        pltpu.sync_copy(zeros_hbm.at[pl.ds(last_off, last_cnt)],
                        agg_sh.at[pl.ds(last_off, last_cnt)])

    plsc.subcore_barrier()

    sd_base = wid * (NB * 2 * B)
    w_base = wid * (NB * B)

    def _stage(b, j):
        pltpu.async_copy(sd_hbm.at[pl.ds(sd_base + b * (2 * B), 2 * B)],
                         sdbuf[j], isem[j])
        pltpu.async_copy(w_hbm.at[pl.ds(w_base + b * B, B)], wbuf[j],
                         isem[j])

    def _stage_wait(b, j):
        pltpu.make_async_copy(
            sd_hbm.at[pl.ds(sd_base + b * (2 * B), 2 * B)], sdbuf[j],
            isem[j]).wait()
        pltpu.make_async_copy(
            w_hbm.at[pl.ds(w_base + b * B, B)], wbuf[j], isem[j]).wait()

    def _gather(j):
        pltpu.async_copy(x_hbm.at[sdbuf[j].at[pl.ds(0, B)]], rows[j],
                         gsem[j])

    def _gather_wait(j):
        pltpu.make_async_copy(x_hbm.at[sdbuf[j].at[pl.ds(0, B)]], rows[j],
                              gsem[j]).wait()

    def _scatter_wait(j):
        pltpu.make_async_copy(rows[j], agg_sh.at[dstc[j]], ssem[j]).wait()

    for j in range(NBUF):
        _stage(j, j)
    _stage_wait(0, 0)
    _gather(0)

    def outer(tt, carry):
        for j in range(NBUF):
            b = tt * NBUF + j
            j2 = 1 - j

            @pl.when((b >= 1) & (b + 1 < NB))
            def _drain_prev(j2=j2):
                _scatter_wait(j2)

            @pl.when(b + 1 < NB)
            def _gather_next(j2=j2, b=b):
                _stage_wait(b + 1, j2)
                _gather(j2)

            _gather_wait(j)

            def group_body(g, carry2, j=j):
                w16 = wbuf[j][pl.ds(g * 16, 16)]
                dstc[j][pl.ds(g * 16, 16)] = (
                    sdbuf[j][pl.ds(B + g * 16, 16)])
                for e16 in range(16):
                    wsp = jnp.full((16,), w16[e16], jnp.float32)
                    for f in range(D // 16):
                        rows[j][g * 16 + e16, pl.ds(f * 16, 16)] = (
                            rows[j][g * 16 + e16, pl.ds(f * 16, 16)] * wsp)
                return carry2

            lax.fori_loop(0, B // 16, group_body, 0)

            pltpu.async_copy(rows[j], agg_sh.at[dstc[j]], ssem[j], add=True)

            @pl.when(b + 2 < NB)
            def _stage_next(j=j, b=b):
                _stage(b + 2, j)
        return carry

    lax.fori_loop(0, NB // NBUF, outer, 0)

    for j in range(NBUF):
        _scatter_wait(j)

    plsc.subcore_barrier()

    @pl.when(s < NS - 1)
    def _out_main():
        pltpu.sync_copy(agg_sh.at[pl.ds(row_off, ROW_CHUNK)],
                        out_hbm.at[c, pl.ds(row_off, ROW_CHUNK)])

    @pl.when(s == NS - 1)
    def _out_last():
        pltpu.sync_copy(agg_sh.at[pl.ds(last_off, last_cnt)],
                        out_hbm.at[c, pl.ds(last_off, last_cnt)])


_sc_spmm = functools.partial(
    pl.kernel,
    out_type=jax.ShapeDtypeStruct((NC, N, D), jnp.float32),
    mesh=plsc.VectorSubcoreMesh(core_axis_name="c", subcore_axis_name="s"),
    scratch_types=(
        [pltpu.VMEM((2 * B,), jnp.int32) for _ in range(NBUF)]
        + [pltpu.VMEM((B,), jnp.float32) for _ in range(NBUF)]
        + [pltpu.VMEM((B,), jnp.int32) for _ in range(NBUF)]
        + [pltpu.VMEM((B, D), jnp.float32) for _ in range(NBUF)]
        + [pltpu.VMEM_SHARED((N, D), jnp.float32)]
        + [pltpu.SemaphoreType.DMA for _ in range(3 * NBUF)]
    ),
)(_sc_spmm_body)


def _tc_body(agg_ref, x_ref, w_ref, sw_ref, bias_ref, gamma_ref, beta_ref,
             out_ref):
    a = agg_ref[0] + agg_ref[1]
    pre = jnp.dot(a, w_ref[...], preferred_element_type=jnp.float32)
    pre = pre + jnp.dot(x_ref[...], sw_ref[...],
                        preferred_element_type=jnp.float32)
    pre = pre + bias_ref[...]
    mean = jnp.mean(pre, axis=0, keepdims=True)
    cen = pre - mean
    var = jnp.mean(cen * cen, axis=0, keepdims=True)
    out_ref[...] = cen * lax.rsqrt(var + 1e-5) * gamma_ref[...] + beta_ref[...]


def kernel(x, edge_weight, weight, self_weight, bias, gamma, beta, edge_index):
    pad = EPAD - E
    dst = jnp.concatenate([edge_index[0], jnp.zeros((pad,), jnp.int32)])
    src = jnp.concatenate([edge_index[1], jnp.zeros((pad,), jnp.int32)])
    w = jnp.concatenate([edge_weight, jnp.zeros((pad,), jnp.float32)])
    sd = jnp.stack([src.reshape(NW * NB, B), dst.reshape(NW * NB, B)],
                   axis=1).reshape(-1)
    zeros = jnp.zeros((N, D), jnp.float32)

    agg = _sc_spmm(sd, w, x, zeros)

    out = pl.pallas_call(
        _tc_body,
        out_shape=jax.ShapeDtypeStruct((N, D), jnp.float32),
    )(agg, x, weight, self_weight,
      bias.reshape(1, D), gamma.reshape(1, D), beta.reshape(1, D))
    return out
